# baseline (speedup 1.0000x reference)
import jax
import jax.numpy as jnp
from jax import lax
from jax.experimental import pallas as pl
from jax.experimental.pallas import tpu as pltpu
from jax.experimental.pallas import tpu_sc as plsc

N = 320000
D = 128
A = 16
G = 1024

NC = 2
NS = 16
NW = NC * NS
ROWS_PER_W = N // NW
CHUNK = 2000
CW = CHUNK // 8
NCHUNKS = ROWS_PER_W // CHUNK
ZROWS = G // NS

TILE = 6400
TILE8 = TILE // 8
N8 = N // 8
D8 = 8 * D


def _proj_body(hg_ref, w1_ref, b1_ref, w2b_ref, b2_ref, qnt_ref, qaw_ref):
    xg = hg_ref[...]
    x = xg.reshape(TILE, D)
    qnt_ref[...] = lax.dot_general(
        w1_ref[...], x, (((0,), (1,)), ((), ())),
        preferred_element_type=jnp.float32) + b1_ref[...]
    qaw_ref[...] = jnp.dot(xg, w2b_ref[...],
                           preferred_element_type=jnp.float32) + b2_ref[...]


_proj = pl.pallas_call(
    _proj_body,
    grid=(N // TILE,),
    in_specs=[
        pl.BlockSpec((TILE8, D8), lambda i: (i, 0)),
        pl.BlockSpec((D, A), lambda i: (0, 0)),
        pl.BlockSpec((A, 1), lambda i: (0, 0)),
        pl.BlockSpec((D8, 8 * A), lambda i: (0, 0)),
        pl.BlockSpec((1, 8 * A), lambda i: (0, 0)),
    ],
    out_specs=[
        pl.BlockSpec((A, TILE), lambda i: (0, i)),
        pl.BlockSpec((TILE8, 8 * A), lambda i: (i, 0)),
    ],
    out_shape=[
        jax.ShapeDtypeStruct((A, N), jnp.float32),
        jax.ShapeDtypeStruct((N8, 8 * A), jnp.float32),
    ],
    compiler_params=pltpu.CompilerParams(
        dimension_semantics=("arbitrary",),
    ),
)


def _segsum_body(qaw_hbm, idx_hbm, out_hbm,
                 rows_v, idx_v, zero_v, id_v, acc_v, acc_sh):
    cid = lax.axis_index("c")
    sid = lax.axis_index("s")
    wid = sid * NC + cid
    iota16 = lax.iota(jnp.int32, 16)
    zeros16 = jnp.zeros((A,), jnp.float32)

    for i in range(ZROWS):
        zero_v[i] = zeros16

    def zero_body(i, carry):
        acc_v[i] = zeros16
        return carry

    lax.fori_loop(0, G, zero_body, 0)
    pltpu.sync_copy(zero_v, acc_sh.at[pl.ds(sid * ZROWS, ZROWS)])
    for j in range(G // 16):
        id_v[pl.ds(j * 16, 16)] = iota16 + (j * 16)
    plsc.subcore_barrier()

    base = wid * ROWS_PER_W
    for k in range(NCHUNKS):
        pltpu.sync_copy(
            qaw_hbm.at[pl.ds((base + k * CHUNK) // 8, CW)], rows_v)
        pltpu.sync_copy(idx_hbm.at[pl.ds(base + k * CHUNK, CHUNK)],
                        idx_v.at[pl.ds(0, CHUNK)])
        idx_v[pl.ds(CHUNK, 16)] = jnp.full((16,), -1, jnp.int32)

        def group_body(gi, running):
            idx16 = idx_v[pl.ds(gi * 16, 16)]
            next16 = idx_v[pl.ds(gi * 16 + 16, 16)]
            for j in range(16):
                row = rows_v[gi * 2 + j // 8, pl.ds((j % 8) * A, A)]
                id_cur = jnp.full((16,), idx16[j], jnp.int32)
                nxt = idx16[j + 1] if j < 15 else next16[0]
                id_next = jnp.full((16,), nxt, jnp.int32)
                bmask = id_cur != id_next
                running = running + row
                plsc.addupdate_scatter(acc_v, [id_cur, iota16], running,
                                       mask=bmask)
                running = jnp.where(bmask, zeros16, running)
            return running

        lax.fori_loop(0, CHUNK // 16, group_body, zeros16)

    pltpu.sync_copy(acc_v, acc_sh.at[id_v], add=True)
    plsc.subcore_barrier()
    pltpu.sync_copy(acc_sh.at[pl.ds(sid * ZROWS, ZROWS)],
                    out_hbm.at[cid, pl.ds(sid * ZROWS, ZROWS)])


_segsum = pl.kernel(
    _segsum_body,
    out_type=jax.ShapeDtypeStruct((NC, G, A), jnp.float32),
    mesh=plsc.VectorSubcoreMesh(core_axis_name="c", subcore_axis_name="s"),
    scratch_types=[
        pltpu.VMEM((CW, 8 * A), jnp.float32),
        pltpu.VMEM((CHUNK + 16,), jnp.int32),
        pltpu.VMEM((ZROWS, A), jnp.float32),
        pltpu.VMEM((G,), jnp.int32),
        pltpu.VMEM((G, A), jnp.float32),
        pltpu.VMEM_SHARED((G, A), jnp.float32),
    ],
    compiler_params=pltpu.CompilerParams(use_tc_tiling_on_sc=False,
                                         needs_layout_passes=False),
)


def _combine_body(p_ref, o_ref):
    o_ref[...] = p_ref[0] + p_ref[1]


_combine = pl.pallas_call(
    _combine_body,
    out_shape=jax.ShapeDtypeStruct((G, A), jnp.float32),
)


def kernel(h_values, q_node_action_w, q_node_action_b, q_action_node_w,
           q_action_node_b, h_indices):
    w2b = jnp.einsum('da,jk->jdka', q_action_node_w,
                     jnp.eye(8, dtype=jnp.float32)).reshape(D8, 8 * A)
    b2t = jnp.tile(q_action_node_b, 8)[None, :]
    hg = h_values.reshape(N8, D8)
    qn_t, qa_w = _proj(hg,
                       q_node_action_w, q_node_action_b[:, None],
                       w2b, b2t)
    partials = _segsum(qa_w, h_indices)
    q_a = _combine(partials)
    return (q_a, qn_t.T)


# speedup vs baseline: 1.1021x; 1.1021x over previous
import jax
import jax.numpy as jnp
from jax import lax
from jax.experimental import pallas as pl
from jax.experimental.pallas import tpu as pltpu
from jax.experimental.pallas import tpu_sc as plsc

N = 320000
D = 128
A = 16
G = 1024

NC = 2
NS = 16
NW = NC * NS
ROWS_PER_W = N // NW
CHUNK = 2000
CW = CHUNK // 8
NCHUNKS = ROWS_PER_W // CHUNK
ZROWS = G // NS

TILE = 6400
TILE8 = TILE // 8
N8 = N // 8
D8 = 8 * D


def _proj_body(hg_ref, w1_ref, b1_ref, w2b_ref, b2_ref, qnt_ref, qaw_ref):
    xg = hg_ref[...]
    x = xg.reshape(TILE, D)
    qnt_ref[...] = lax.dot_general(
        w1_ref[...], x, (((0,), (1,)), ((), ())),
        preferred_element_type=jnp.float32) + b1_ref[...]
    qaw_ref[...] = jnp.dot(xg, w2b_ref[...],
                           preferred_element_type=jnp.float32) + b2_ref[...]


_proj = pl.pallas_call(
    _proj_body,
    grid=(N // TILE,),
    in_specs=[
        pl.BlockSpec((TILE8, D8), lambda i: (i, 0)),
        pl.BlockSpec((D, A), lambda i: (0, 0)),
        pl.BlockSpec((A, 1), lambda i: (0, 0)),
        pl.BlockSpec((D8, 8 * A), lambda i: (0, 0)),
        pl.BlockSpec((1, 8 * A), lambda i: (0, 0)),
    ],
    out_specs=[
        pl.BlockSpec((A, TILE), lambda i: (0, i)),
        pl.BlockSpec((TILE8, 8 * A), lambda i: (i, 0)),
    ],
    out_shape=[
        jax.ShapeDtypeStruct((A, N), jnp.float32),
        jax.ShapeDtypeStruct((N8, 8 * A), jnp.float32),
    ],
    compiler_params=pltpu.CompilerParams(
        dimension_semantics=("arbitrary",),
    ),
)


def _segsum_body(rows_hbm, idx_hbm, out_hbm, rows_v, idx_v, zero_v, acc_sh):
    cid = lax.axis_index("c")
    sid = lax.axis_index("s")
    wid = sid * NC + cid
    for i in range(ZROWS):
        zero_v[i] = jnp.zeros((A,), jnp.float32)
    pltpu.sync_copy(zero_v, acc_sh.at[pl.ds(sid * ZROWS, ZROWS)])
    plsc.subcore_barrier()
    base = wid * ROWS_PER_W
    for k in range(NCHUNKS):
        pltpu.sync_copy(rows_hbm.at[pl.ds(base + k * CHUNK, CHUNK)], rows_v)
        pltpu.sync_copy(idx_hbm.at[pl.ds(base + k * CHUNK, CHUNK)], idx_v)
        pltpu.sync_copy(rows_v, acc_sh.at[idx_v], add=True)
    plsc.subcore_barrier()
    pltpu.sync_copy(acc_sh.at[pl.ds(sid * ZROWS, ZROWS)],
                    out_hbm.at[cid, pl.ds(sid * ZROWS, ZROWS)])


_segsum = pl.kernel(
    _segsum_body,
    out_type=jax.ShapeDtypeStruct((NC, G, A), jnp.float32),
    mesh=plsc.VectorSubcoreMesh(core_axis_name="c", subcore_axis_name="s"),
    scratch_types=[
        pltpu.VMEM((CHUNK, A), jnp.float32),
        pltpu.VMEM((CHUNK,), jnp.int32),
        pltpu.VMEM((ZROWS, A), jnp.float32),
        pltpu.VMEM_SHARED((G, A), jnp.float32),
    ],
    compiler_params=pltpu.CompilerParams(use_tc_tiling_on_sc=False),
)


def _combine_body(p_ref, o_ref):
    o_ref[...] = p_ref[0] + p_ref[1]


_combine = pl.pallas_call(
    _combine_body,
    out_shape=jax.ShapeDtypeStruct((G, A), jnp.float32),
)


def kernel(h_values, q_node_action_w, q_node_action_b, q_action_node_w,
           q_action_node_b, h_indices):
    w2b = jnp.einsum('da,jk->jdka', q_action_node_w,
                     jnp.eye(8, dtype=jnp.float32)).reshape(D8, 8 * A)
    b2t = jnp.tile(q_action_node_b, 8)[None, :]
    hg = h_values.reshape(N8, D8)
    qn_t, qa_w = _proj(hg,
                       q_node_action_w, q_node_action_b[:, None],
                       w2b, b2t)
    partials = _segsum(qa_w.reshape(N, A), h_indices)
    q_a = _combine(partials)
    return (q_a, qn_t.T)


# speedup vs baseline: 1.2584x; 1.1418x over previous
import jax
import jax.numpy as jnp
from jax import lax
from jax.experimental import pallas as pl
from jax.experimental.pallas import tpu as pltpu
from jax.experimental.pallas import tpu_sc as plsc

N = 320000
D = 128
A = 16
G = 1024

NC = 2
NS = 16
NW = NC * NS
ROWS_PER_W = N // NW
CHUNK = 5000
NCHUNKS = ROWS_PER_W // CHUNK
ZROWS = G // NS

TILE = 6400


def _proj_body(h_ref, w1_ref, b1_ref, w2_ref, b2_ref, qnt_ref, qat_ref):
    x = h_ref[...]
    qnt_ref[...] = lax.dot_general(
        w1_ref[...], x, (((0,), (1,)), ((), ())),
        preferred_element_type=jnp.float32) + b1_ref[...]
    qat_ref[...] = jnp.dot(x, w2_ref[...],
                           preferred_element_type=jnp.float32) + b2_ref[...]


_proj = pl.pallas_call(
    _proj_body,
    grid=(N // TILE,),
    in_specs=[
        pl.BlockSpec((TILE, D), lambda i: (i, 0)),
        pl.BlockSpec((D, A), lambda i: (0, 0)),
        pl.BlockSpec((A, 1), lambda i: (0, 0)),
        pl.BlockSpec((D, A), lambda i: (0, 0)),
        pl.BlockSpec((1, A), lambda i: (0, 0)),
    ],
    out_specs=[
        pl.BlockSpec((A, TILE), lambda i: (0, i)),
        pl.BlockSpec((TILE, A), lambda i: (i, 0)),
    ],
    out_shape=[
        jax.ShapeDtypeStruct((A, N), jnp.float32),
        jax.ShapeDtypeStruct((N, A), jnp.float32),
    ],
    compiler_params=pltpu.CompilerParams(
        dimension_semantics=("arbitrary",),
    ),
)


def _segsum_body(rows_hbm, idx_hbm, out_hbm, rows_v, idx_v, zero_v, acc_sh):
    cid = lax.axis_index("c")
    sid = lax.axis_index("s")
    wid = sid * NC + cid
    for i in range(ZROWS):
        zero_v[i] = jnp.zeros((A,), jnp.float32)
    pltpu.sync_copy(zero_v, acc_sh.at[pl.ds(sid * ZROWS, ZROWS)])
    plsc.subcore_barrier()
    base = wid * ROWS_PER_W
    for k in range(NCHUNKS):
        pltpu.sync_copy(rows_hbm.at[pl.ds(base + k * CHUNK, CHUNK)], rows_v)
        pltpu.sync_copy(idx_hbm.at[pl.ds(base + k * CHUNK, CHUNK)], idx_v)
        pltpu.sync_copy(rows_v, acc_sh.at[idx_v], add=True)
    plsc.subcore_barrier()
    pltpu.sync_copy(acc_sh.at[pl.ds(sid * ZROWS, ZROWS)],
                    out_hbm.at[cid, pl.ds(sid * ZROWS, ZROWS)])


_segsum = pl.kernel(
    _segsum_body,
    out_type=jax.ShapeDtypeStruct((NC, G, A), jnp.float32),
    mesh=plsc.VectorSubcoreMesh(core_axis_name="c", subcore_axis_name="s"),
    scratch_types=[
        pltpu.VMEM((CHUNK, A), jnp.float32),
        pltpu.VMEM((CHUNK,), jnp.int32),
        pltpu.VMEM((ZROWS, A), jnp.float32),
        pltpu.VMEM_SHARED((G, A), jnp.float32),
    ],
    compiler_params=pltpu.CompilerParams(use_tc_tiling_on_sc=False),
)


def _combine_body(p_ref, o_ref):
    o_ref[...] = p_ref[0] + p_ref[1]


_combine = pl.pallas_call(
    _combine_body,
    out_shape=jax.ShapeDtypeStruct((G, A), jnp.float32),
)


def kernel(h_values, q_node_action_w, q_node_action_b, q_action_node_w,
           q_action_node_b, h_indices):
    qn_t, qa_n = _proj(h_values,
                       q_node_action_w, q_node_action_b[:, None],
                       q_action_node_w, q_action_node_b[None, :])
    partials = _segsum(qa_n, h_indices)
    q_a = _combine(partials)
    return (q_a, qn_t.T)
